# baseline (speedup 1.0000x reference)
import jax
import jax.numpy as jnp
from jax import lax
from jax.experimental import pallas as pl
from jax.experimental.pallas import tpu as pltpu
from jax.experimental.pallas import tpu_sc as plsc

N_NODES = 10000
N_EDGES = 320000
D_NODE = 128
D_EDGE = 16

NC = 2
NS = 16
NW = NC * NS
GRP_EDGES = 1024
GROUPS = 10
EDGES_PER_TILE = GRP_EDGES * GROUPS
E_PAD = NW * EDGES_PER_TILE
PACK = 128 // D_EDGE
ROWS = N_EDGES // PACK


def _pq_body(ns_ref, wa_ref, wb_ref, t_ref):
    half = pl.num_programs(0) // 2
    t = pl.program_id(0)
    w = jnp.where(t < half, wa_ref[...], wb_ref[...])
    t_ref[...] = jnp.dot(ns_ref[...], w, preferred_element_type=jnp.float32)


def _precompute_table(ns, wa, wb):
    br = 2000
    nb = N_NODES // br
    return pl.pallas_call(
        _pq_body,
        grid=(2 * nb,),
        in_specs=[
            pl.BlockSpec((br, D_NODE), lambda t: (t % (N_NODES // 2000), 0)),
            pl.BlockSpec((D_NODE, D_EDGE), lambda t: (0, 0)),
            pl.BlockSpec((D_NODE, D_EDGE), lambda t: (0, 0)),
        ],
        out_specs=pl.BlockSpec((br, D_EDGE), lambda t: (t, 0)),
        out_shape=jax.ShapeDtypeStruct((2 * N_NODES, D_EDGE), jnp.float32),
    )(ns, wa, wb)


E_TILE = N_EDGES // NW
EG = 1000
ER = EG // PACK
R_TILE = E_TILE // PACK


def _gather_body(t_hbm, cidx_hbm, e_hbm, s_hbm, epk_hbm,
                 idx_v, a0, a1, b0, b1, c0, c1, gsem, wsem):
    wid = lax.axis_index("s") * NC + lax.axis_index("c")
    pltpu.sync_copy(cidx_hbm.at[wid], idx_v)
    base = wid * EDGES_PER_TILE
    ab = ((a0, b0), (a1, b1))
    cb = (c0, c1)

    def g_pairs(g, par):
        sl = pl.ds(g * GRP_EDGES, GRP_EDGES)
        return ((t_hbm.at[idx_v.at[0, sl]], ab[par][0], gsem),
                (t_hbm.at[idx_v.at[1, sl]], ab[par][1], gsem))

    def w_pair(g, par):
        r0 = (base + g * GRP_EDGES) // PACK
        return ((cb[par], s_hbm.at[pl.ds(r0, GRP_EDGES // PACK)], wsem),)

    def fire(pairs):
        for s, d, sem in pairs:
            pltpu.async_copy(s, d, sem)

    def drain(pairs):
        for s, d, sem in pairs:
            pltpu.make_async_copy(s, d, sem).wait()

    fire(g_pairs(0, 0))
    for g in range(GROUPS):
        par = g & 1
        if g + 1 < GROUPS:
            fire(g_pairs(g + 1, 1 - par))
        drain(g_pairs(g, par))
        if g >= 2:
            drain(w_pair(g - 2, par))
        a, b = ab[par]
        c = cb[par]

        @pl.loop(0, GRP_EDGES // PACK, unroll=2)
        def _add(r):
            for k in range(PACK):
                e_i = r * PACK + k
                c[r, 16 * k:16 * (k + 1)] = a[e_i, :] + b[e_i, :]

        fire(w_pair(g, par))
    drain(w_pair(GROUPS - 2, (GROUPS - 2) & 1))
    drain(w_pair(GROUPS - 1, (GROUPS - 1) & 1))

    base_e = wid * E_TILE
    base_r = wid * R_TILE
    ng = E_TILE // EG

    def e_r(g, par):
        return ((e_hbm.at[pl.ds(base_e + g * EG, EG)],
                 ab[par][0].at[pl.ds(0, EG)], gsem),)

    def e_w(g, par):
        return ((cb[par].at[pl.ds(0, ER)],
                 epk_hbm.at[pl.ds(base_r + g * ER, ER)], wsem),)

    fire(e_r(0, 0))
    for g in range(ng):
        par = g & 1
        if g + 1 < ng:
            fire(e_r(g + 1, 1 - par))
        drain(e_r(g, par))
        if g >= 2:
            drain(e_w(g - 2, par))
        a = ab[par][0]
        c = cb[par]

        @pl.loop(0, ER, unroll=2)
        def _rp(r):
            for k in range(PACK):
                c[r, 16 * k:16 * (k + 1)] = a[r * PACK + k, :]

        fire(e_w(g, par))
    drain(e_w(ng - 2, (ng - 2) & 1))
    drain(e_w(ng - 1, (ng - 1) & 1))


def _gather_add(table, cidx, edge_feats):
    mesh = plsc.VectorSubcoreMesh(core_axis_name="c", subcore_axis_name="s")
    buf = pltpu.VMEM((GRP_EDGES, D_EDGE), jnp.float32)
    cbuf = pltpu.VMEM((GRP_EDGES // PACK, 128), jnp.float32)
    f = pl.kernel(
        _gather_body,
        out_type=(jax.ShapeDtypeStruct((E_PAD // PACK, 128), jnp.float32),
                  jax.ShapeDtypeStruct((ROWS, 128), jnp.float32)),
        mesh=mesh,
        scratch_types=[
            pltpu.VMEM((2, EDGES_PER_TILE), jnp.int32),
            buf, buf, buf, buf, cbuf, cbuf,
            pltpu.SemaphoreType.DMA,
            pltpu.SemaphoreType.DMA,
        ],
        compiler_params=pltpu.CompilerParams(use_tc_tiling_on_sc=False),
    )
    return f(table, cidx, edge_feats)


def _unpack_body(opk_hbm, out_hbm, r0b, r1b, u0, u1, gsem, wsem):
    wid = lax.axis_index("s") * NC + lax.axis_index("c")
    base_e = wid * E_TILE
    base_r = wid * R_TILE
    rb = (r0b, r1b)
    ub = (u0, u1)
    ng = E_TILE // EG

    def o_r(g, par):
        return ((opk_hbm.at[pl.ds(base_r + g * ER, ER)], rb[par], gsem),)

    def o_w(g, par):
        return ((ub[par], out_hbm.at[pl.ds(base_e + g * EG, EG)], wsem),)

    def fire(pairs):
        for s, d, sem in pairs:
            pltpu.async_copy(s, d, sem)

    def drain(pairs):
        for s, d, sem in pairs:
            pltpu.make_async_copy(s, d, sem).wait()

    fire(o_r(0, 0))
    for g in range(ng):
        par = g & 1
        if g + 1 < ng:
            fire(o_r(g + 1, 1 - par))
        drain(o_r(g, par))
        if g >= 2:
            drain(o_w(g - 2, par))
        r_ = rb[par]
        u = ub[par]

        @pl.loop(0, ER, unroll=2)
        def _up(r):
            for k in range(PACK):
                u[r * PACK + k, :] = r_[r, 16 * k:16 * (k + 1)]

        fire(o_w(g, par))
    drain(o_w(ng - 2, (ng - 2) & 1))
    drain(o_w(ng - 1, (ng - 1) & 1))


def _unpack(opk):
    mesh = plsc.VectorSubcoreMesh(core_axis_name="c", subcore_axis_name="s")
    f = pl.kernel(
        _unpack_body,
        out_type=jax.ShapeDtypeStruct((N_EDGES, D_EDGE), jnp.float32),
        mesh=mesh,
        scratch_types=[
            pltpu.VMEM((ER, 128), jnp.float32),
            pltpu.VMEM((ER, 128), jnp.float32),
            pltpu.VMEM((EG, D_EDGE), jnp.float32),
            pltpu.VMEM((EG, D_EDGE), jnp.float32),
            pltpu.SemaphoreType.DMA,
            pltpu.SemaphoreType.DMA,
        ],
        compiler_params=pltpu.CompilerParams(use_tc_tiling_on_sc=False),
    )
    return f(opk)


def _dense_body(s_ref, e_ref, w1_ref, w2_ref, ma_ref, pr_ref, o_ref):
    e = e_ref[...]
    x = (s_ref[...]
         + jnp.dot(e, w1_ref[...], preferred_element_type=jnp.float32)
         + pr_ref[0:1, :])
    h1 = x * (1.0 / (1.0 + jnp.exp(-x)))
    y = jnp.dot(h1, w2_ref[...], preferred_element_type=jnp.float32) + pr_ref[1:2, :]
    h2 = y * (1.0 / (1.0 + jnp.exp(-y)))
    z = e + h2
    m = jnp.dot(z, ma_ref[...], preferred_element_type=jnp.float32)
    s2 = jnp.dot(z * z, ma_ref[...], preferred_element_type=jnp.float32)
    var = s2 - m * m
    o_ref[...] = (z - m) * lax.rsqrt(var + 1e-5) * pr_ref[2:3, :] + pr_ref[3:4, :]


def _dense(s_pk, e_pk, w1blk, w2blk, mavg, params):
    br = 2000
    full = lambda t: (0, 0)
    row = lambda t: (t, 0)
    return pl.pallas_call(
        _dense_body,
        grid=(ROWS // br,),
        in_specs=[
            pl.BlockSpec((br, 128), row),
            pl.BlockSpec((br, 128), row),
            pl.BlockSpec((128, 128), full),
            pl.BlockSpec((128, 128), full),
            pl.BlockSpec((128, 128), full),
            pl.BlockSpec((8, 128), full),
        ],
        out_specs=pl.BlockSpec((br, 128), row),
        out_shape=jax.ShapeDtypeStruct((ROWS, 128), jnp.float32),
    )(s_pk, e_pk, w1blk, w2blk, mavg, params)


def kernel(node_scalars, edge_index, edge_feats, W1, b1, W2, b2, gamma, beta):
    wa = W1[:D_NODE]
    wb = W1[D_NODE:2 * D_NODE]
    we = W1[2 * D_NODE:]

    table = _precompute_table(node_scalars, wa, wb)

    pad = E_PAD - N_EDGES
    src = jnp.pad(edge_index[0].astype(jnp.int32), (0, pad))
    dst = jnp.pad(edge_index[1].astype(jnp.int32), (0, pad)) + N_NODES
    cidx = jnp.stack([src.reshape(NW, EDGES_PER_TILE),
                      dst.reshape(NW, EDGES_PER_TILE)], axis=1)

    s_pk, e_pk = _gather_add(table, cidx, edge_feats)

    eye = jnp.eye(PACK, dtype=jnp.float32)
    w1blk = jnp.kron(eye, we)
    w2blk = jnp.kron(eye, W2)
    mavg = jnp.kron(eye, jnp.full((D_EDGE, D_EDGE), 1.0 / D_EDGE, jnp.float32))
    params = jnp.concatenate([
        jnp.tile(b1, PACK)[None],
        jnp.tile(b2, PACK)[None],
        jnp.tile(gamma, PACK)[None],
        jnp.tile(beta, PACK)[None],
        jnp.zeros((4, 128), jnp.float32),
    ], axis=0)

    out_pk = _dense(s_pk, e_pk, w1blk, w2blk, mavg, params)
    return _unpack(out_pk)


# speedup vs baseline: 1.2276x; 1.2276x over previous
import jax
import jax.numpy as jnp
from jax import lax
from jax.experimental import pallas as pl
from jax.experimental.pallas import tpu as pltpu
from jax.experimental.pallas import tpu_sc as plsc

N_NODES = 10000
N_EDGES = 320000
D_NODE = 128
D_EDGE = 16

NC = 2
NS = 16
NW = NC * NS
GRP_EDGES = 1024
GROUPS = 10
EDGES_PER_TILE = GRP_EDGES * GROUPS
E_PAD = NW * EDGES_PER_TILE
PACK = 128 // D_EDGE
ROWS = N_EDGES // PACK


def _pq_body(ns_ref, wa_ref, wb_ref, t_ref):
    half = pl.num_programs(0) // 2
    t = pl.program_id(0)
    w = jnp.where(t < half, wa_ref[...], wb_ref[...])
    t_ref[...] = jnp.dot(ns_ref[...], w, preferred_element_type=jnp.float32)


def _precompute_table(ns, wa, wb):
    br = 2000
    nb = N_NODES // br
    return pl.pallas_call(
        _pq_body,
        grid=(2 * nb,),
        in_specs=[
            pl.BlockSpec((br, D_NODE), lambda t: (t % (N_NODES // 2000), 0)),
            pl.BlockSpec((D_NODE, D_EDGE), lambda t: (0, 0)),
            pl.BlockSpec((D_NODE, D_EDGE), lambda t: (0, 0)),
        ],
        out_specs=pl.BlockSpec((br, D_EDGE), lambda t: (t, 0)),
        out_shape=jax.ShapeDtypeStruct((2 * N_NODES, D_EDGE), jnp.float32),
    )(ns, wa, wb)


E_TILE = N_EDGES // NW
EG = 1000
ER = EG // PACK
R_TILE = E_TILE // PACK


def _gather_body(t_hbm, cidx_hbm, s_hbm,
                 idx_v, a0, a1, b0, b1, c0, c1, gsem, wsem):
    wid = lax.axis_index("s") * NC + lax.axis_index("c")
    pltpu.sync_copy(cidx_hbm.at[wid], idx_v)
    base = wid * EDGES_PER_TILE
    ab = ((a0, b0), (a1, b1))
    cb = (c0, c1)

    def g_pairs(g, par):
        sl = pl.ds(g * GRP_EDGES, GRP_EDGES)
        return ((t_hbm.at[idx_v.at[0, sl]], ab[par][0], gsem),
                (t_hbm.at[idx_v.at[1, sl]], ab[par][1], gsem))

    def w_pair(g, par):
        r0 = (base + g * GRP_EDGES) // PACK
        return ((cb[par], s_hbm.at[pl.ds(r0, GRP_EDGES // PACK)], wsem),)

    def fire(pairs):
        for s, d, sem in pairs:
            pltpu.async_copy(s, d, sem)

    def drain(pairs):
        for s, d, sem in pairs:
            pltpu.make_async_copy(s, d, sem).wait()

    fire(g_pairs(0, 0))
    for g in range(GROUPS):
        par = g & 1
        if g + 1 < GROUPS:
            fire(g_pairs(g + 1, 1 - par))
        drain(g_pairs(g, par))
        if g >= 2:
            drain(w_pair(g - 2, par))
        a, b = ab[par]
        c = cb[par]

        @pl.loop(0, GRP_EDGES // PACK, unroll=2)
        def _add(r):
            for k in range(PACK):
                e_i = r * PACK + k
                c[r, 16 * k:16 * (k + 1)] = a[e_i, :] + b[e_i, :]

        fire(w_pair(g, par))
    drain(w_pair(GROUPS - 2, (GROUPS - 2) & 1))
    drain(w_pair(GROUPS - 1, (GROUPS - 1) & 1))


def _gather_add(table, cidx):
    mesh = plsc.VectorSubcoreMesh(core_axis_name="c", subcore_axis_name="s")
    buf = pltpu.VMEM((GRP_EDGES, D_EDGE), jnp.float32)
    cbuf = pltpu.VMEM((GRP_EDGES // PACK, 128), jnp.float32)
    f = pl.kernel(
        _gather_body,
        out_type=jax.ShapeDtypeStruct((E_PAD // PACK, 128), jnp.float32),
        mesh=mesh,
        scratch_types=[
            pltpu.VMEM((2, EDGES_PER_TILE), jnp.int32),
            buf, buf, buf, buf, cbuf, cbuf,
            pltpu.SemaphoreType.DMA,
            pltpu.SemaphoreType.DMA,
        ],
        compiler_params=pltpu.CompilerParams(use_tc_tiling_on_sc=False),
    )
    return f(table, cidx)


def _dense_body(s_ref, e_ref, w1_ref, w2_ref, ma_ref, pr_ref, o_ref):
    br = s_ref.shape[0]
    e = jnp.concatenate([e_ref[k::PACK, :] for k in range(PACK)], axis=1)
    x = (s_ref[...]
         + jnp.dot(e, w1_ref[...], preferred_element_type=jnp.float32)
         + pr_ref[0:1, :])
    h1 = x * (1.0 / (1.0 + jnp.exp(-x)))
    y = jnp.dot(h1, w2_ref[...], preferred_element_type=jnp.float32) + pr_ref[1:2, :]
    h2 = y * (1.0 / (1.0 + jnp.exp(-y)))
    z = e + h2
    m = jnp.dot(z, ma_ref[...], preferred_element_type=jnp.float32)
    s2 = jnp.dot(z * z, ma_ref[...], preferred_element_type=jnp.float32)
    var = s2 - m * m
    out = (z - m) * lax.rsqrt(var + 1e-5) * pr_ref[2:3, :] + pr_ref[3:4, :]
    for k in range(PACK):
        o_ref[k::PACK, :] = out[:, 16 * k:16 * (k + 1)]


def _dense(s_pk, e_nat, w1blk, w2blk, mavg, params):
    br = 2000
    full = lambda t: (0, 0)
    row = lambda t: (t, 0)
    return pl.pallas_call(
        _dense_body,
        grid=(ROWS // br,),
        in_specs=[
            pl.BlockSpec((br, 128), row),
            pl.BlockSpec((br * PACK, D_EDGE), row),
            pl.BlockSpec((128, 128), full),
            pl.BlockSpec((128, 128), full),
            pl.BlockSpec((128, 128), full),
            pl.BlockSpec((8, 128), full),
        ],
        out_specs=pl.BlockSpec((br * PACK, D_EDGE), row),
        out_shape=jax.ShapeDtypeStruct((N_EDGES, D_EDGE), jnp.float32),
    )(s_pk, e_nat, w1blk, w2blk, mavg, params)


def kernel(node_scalars, edge_index, edge_feats, W1, b1, W2, b2, gamma, beta):
    wa = W1[:D_NODE]
    wb = W1[D_NODE:2 * D_NODE]
    we = W1[2 * D_NODE:]

    table = _precompute_table(node_scalars, wa, wb)

    pad = E_PAD - N_EDGES
    src = jnp.pad(edge_index[0].astype(jnp.int32), (0, pad))
    dst = jnp.pad(edge_index[1].astype(jnp.int32), (0, pad)) + N_NODES
    cidx = jnp.stack([src.reshape(NW, EDGES_PER_TILE),
                      dst.reshape(NW, EDGES_PER_TILE)], axis=1)

    s_pk = _gather_add(table, cidx)

    eye = jnp.eye(PACK, dtype=jnp.float32)
    w1blk = jnp.kron(eye, we)
    w2blk = jnp.kron(eye, W2)
    mavg = jnp.kron(eye, jnp.full((D_EDGE, D_EDGE), 1.0 / D_EDGE, jnp.float32))
    params = jnp.concatenate([
        jnp.tile(b1, PACK)[None],
        jnp.tile(b2, PACK)[None],
        jnp.tile(gamma, PACK)[None],
        jnp.tile(beta, PACK)[None],
        jnp.zeros((4, 128), jnp.float32),
    ], axis=0)

    return _dense(s_pk, edge_feats, w1blk, w2blk, mavg, params)


# speedup vs baseline: 2.0808x; 1.6950x over previous
import jax
import jax.numpy as jnp
from jax import lax
from jax.experimental import pallas as pl
from jax.experimental.pallas import tpu as pltpu
from jax.experimental.pallas import tpu_sc as plsc

N_NODES = 10000
N_EDGES = 320000
D_NODE = 128
D_EDGE = 16

NC = 2
NS = 16
NW = NC * NS
GRP_EDGES = 1024
GROUPS = 10
EDGES_PER_TILE = GRP_EDGES * GROUPS
E_PAD = NW * EDGES_PER_TILE


def _pq_body(ns_ref, wa_ref, wb_ref, t_ref):
    half = pl.num_programs(0) // 2
    t = pl.program_id(0)
    w = jnp.where(t < half, wa_ref[...], wb_ref[...])
    t_ref[...] = jnp.dot(ns_ref[...], w, preferred_element_type=jnp.float32)


def _precompute_table(ns, wa, wb):
    br = 2000
    nb = N_NODES // br
    return pl.pallas_call(
        _pq_body,
        grid=(2 * nb,),
        in_specs=[
            pl.BlockSpec((br, D_NODE), lambda t: (t % (N_NODES // 2000), 0)),
            pl.BlockSpec((D_NODE, D_EDGE), lambda t: (0, 0)),
            pl.BlockSpec((D_NODE, D_EDGE), lambda t: (0, 0)),
        ],
        out_specs=pl.BlockSpec((br, D_EDGE), lambda t: (t, 0)),
        out_shape=jax.ShapeDtypeStruct((2 * N_NODES, D_EDGE), jnp.float32),
    )(ns, wa, wb)


def _gather_body(t_hbm, cidx_hbm, st_hbm,
                 idx_v, a0, a1, b0, b1, c0, c1, gsem, wsem):
    wid = lax.axis_index("s") * NC + lax.axis_index("c")
    pltpu.sync_copy(cidx_hbm.at[wid], idx_v)
    base = wid * EDGES_PER_TILE
    ab = ((a0, b0), (a1, b1))
    cb = (c0, c1)
    iota16 = lax.iota(jnp.int32, 16)

    def g_pairs(g, par):
        sl = pl.ds(g * GRP_EDGES, GRP_EDGES)
        return ((t_hbm.at[idx_v.at[0, sl]], ab[par][0], gsem),
                (t_hbm.at[idx_v.at[1, sl]], ab[par][1], gsem))

    def w_pair(g, par):
        e0 = base + g * GRP_EDGES
        return ((cb[par], st_hbm.at[:, pl.ds(e0, GRP_EDGES)], wsem),)

    def fire(pairs):
        for s, d, sem in pairs:
            pltpu.async_copy(s, d, sem)

    def drain(pairs):
        for s, d, sem in pairs:
            pltpu.make_async_copy(s, d, sem).wait()

    fire(g_pairs(0, 0))
    for g in range(GROUPS):
        par = g & 1
        if g + 1 < GROUPS:
            fire(g_pairs(g + 1, 1 - par))
        drain(g_pairs(g, par))
        if g >= 2:
            drain(w_pair(g - 2, par))
        a, b = ab[par]
        c = cb[par]

        @pl.loop(0, GRP_EDGES // 16)
        def _addt(eb):
            rows = eb * 16 + iota16
            for j in range(D_EDGE):
                col = jnp.full((16,), j, jnp.int32)
                va = plsc.load_gather(a, [rows, col])
                vb = plsc.load_gather(b, [rows, col])
                c[j, pl.ds(eb * 16, 16)] = va + vb

        fire(w_pair(g, par))
    drain(w_pair(GROUPS - 2, (GROUPS - 2) & 1))
    drain(w_pair(GROUPS - 1, (GROUPS - 1) & 1))


def _gather_add(table, cidx):
    mesh = plsc.VectorSubcoreMesh(core_axis_name="c", subcore_axis_name="s")
    buf = pltpu.VMEM((GRP_EDGES, D_EDGE), jnp.float32)
    cbuf = pltpu.VMEM((D_EDGE, GRP_EDGES), jnp.float32)
    f = pl.kernel(
        _gather_body,
        out_type=jax.ShapeDtypeStruct((D_EDGE, E_PAD), jnp.float32),
        mesh=mesh,
        scratch_types=[
            pltpu.VMEM((2, EDGES_PER_TILE), jnp.int32),
            buf, buf, buf, buf, cbuf, cbuf,
            pltpu.SemaphoreType.DMA,
            pltpu.SemaphoreType.DMA,
        ],
        compiler_params=pltpu.CompilerParams(use_tc_tiling_on_sc=False,
                                             needs_layout_passes=False),
    )
    return f(table, cidx)


def _dense_body(st_ref, et_ref, w1t_ref, w2t_ref, pr_ref, o_ref):
    e = et_ref[...]
    x = (st_ref[...]
         + jnp.dot(w1t_ref[...], e, preferred_element_type=jnp.float32)
         + pr_ref[:, 0:1])
    h1 = x * (1.0 / (1.0 + jnp.exp(-x)))
    y = jnp.dot(w2t_ref[...], h1, preferred_element_type=jnp.float32) + pr_ref[:, 1:2]
    h2 = y * (1.0 / (1.0 + jnp.exp(-y)))
    z = e + h2
    m = jnp.mean(z, axis=0, keepdims=True)
    v = jnp.mean(z * z, axis=0, keepdims=True) - m * m
    o_ref[...] = (z - m) * lax.rsqrt(v + 1e-5) * pr_ref[:, 2:3] + pr_ref[:, 3:4]


def _dense(st, et, w1t, w2t, params_t):
    bc = 32000
    full = lambda t: (0, 0)
    col = lambda t: (0, t)
    return pl.pallas_call(
        _dense_body,
        grid=(N_EDGES // bc,),
        in_specs=[
            pl.BlockSpec((D_EDGE, bc), col),
            pl.BlockSpec((D_EDGE, bc), col),
            pl.BlockSpec((D_EDGE, D_EDGE), full),
            pl.BlockSpec((D_EDGE, D_EDGE), full),
            pl.BlockSpec((D_EDGE, 8), full),
        ],
        out_specs=pl.BlockSpec((D_EDGE, bc), col),
        out_shape=jax.ShapeDtypeStruct((D_EDGE, N_EDGES), jnp.float32),
    )(st, et, w1t, w2t, params_t)


def kernel(node_scalars, edge_index, edge_feats, W1, b1, W2, b2, gamma, beta):
    wa = W1[:D_NODE]
    wb = W1[D_NODE:2 * D_NODE]
    we = W1[2 * D_NODE:]

    table = _precompute_table(node_scalars, wa, wb)

    pad = E_PAD - N_EDGES
    src = jnp.pad(edge_index[0].astype(jnp.int32), (0, pad))
    dst = jnp.pad(edge_index[1].astype(jnp.int32), (0, pad)) + N_NODES
    cidx = jnp.stack([src.reshape(NW, EDGES_PER_TILE),
                      dst.reshape(NW, EDGES_PER_TILE)], axis=1)

    st = _gather_add(table, cidx)

    params_t = jnp.stack(
        [b1, b2, gamma, beta] + [jnp.zeros_like(b1)] * 4, axis=1)
    out_t = _dense(st, edge_feats.T, we.T, W2.T, params_t)
    return out_t.T
